# baseline (speedup 1.0000x reference)
import functools

import jax
import jax.numpy as jnp
from jax.experimental import pallas as pl
from jax.experimental.pallas import tpu as pltpu

_NEG_BIG = -3e38
_NEG_REF = -1e9


def _net_step_kernel(lengths_ref, x_ref, adj_ref, hints_ref, we_in_ref,
                     we_hint_ref, w_edge_ref, wcat_ref, wo2_ref, wdec_ref,
                     out_ref, base_ref):
    N = adj_ref.shape[1]
    H = wo2_ref.shape[1]

    b = pl.program_id(0)
    L = lengths_ref[b]
    n_steps = jnp.maximum(L - 1, 1)

    x = x_ref[0, 0, :]
    wedge = w_edge_ref[0, :]
    wdec = wdec_ref[0, :]
    wcat_a, wcat_b = wcat_ref[:H, :], wcat_ref[H:, :]
    wo2 = wo2_ref[...]

    dot = functools.partial(jnp.dot, preferred_element_type=jnp.float32)
    u_wa = dot(we_in_ref[...], wcat_a)
    v_wa = dot(we_hint_ref[...], wcat_a)

    def enc_mm(hint_i):
        hb = jnp.broadcast_to(hint_i[:, None], (N, H))
        return xu + jnp.concatenate(
            [hb * v_wa[:, :H], hb * v_wa[:, H:2 * H], hb * v_wa[:, 2 * H:]],
            axis=1)

    adj = adj_ref[0]
    anycol = jnp.max(adj.T, axis=1, keepdims=True) > 0.5
    xu = x[:, None] * u_wa

    def decode(mm, msgs, is_last):
        o1 = mm[:, 2 * H:]
        h_new = jnp.maximum(o1 + dot(msgs, wo2), 0.0)

        @pl.when(is_last)
        def _():
            out_ref[0, 0, :] = jnp.sum(h_new * wdec[None, :], axis=1)
        return h_new

    hint0 = hints_ref[0, 0:1, :][0]
    mm0 = enc_mm(hint0)
    m1_0 = mm0[:, :H]
    g = None
    for k in range(0, N, 16):
        a3 = adj[k:k + 16, :][:, :, None]
        blk = jnp.where(a3 > 0.5, a3 * wedge[None, None, :], _NEG_BIG)
        base_ref[k:k + 16] = blk
        for s in range(16):
            cand = m1_0[k + s:k + s + 1, :] + blk[s]
            g = cand if g is None else jnp.maximum(g, cand)
    msgs0 = jnp.where(anycol, jnp.maximum(g + mm0[:, H:2 * H], 0.0), _NEG_REF)
    h0 = decode(mm0, msgs0, n_steps == 1)

    def step(i, hidden):
        hint_i = hints_ref[0, pl.ds(i, 1), :][0]
        mm = enc_mm(hint_i) + dot(hidden, wcat_b)
        m1 = mm[:, :H]
        g = m1[0:1, :] + base_ref[0]
        for i_src in range(1, N):
            g = jnp.maximum(g, m1[i_src:i_src + 1, :] + base_ref[i_src])
        msgs = jnp.where(anycol, jnp.maximum(g + mm[:, H:2 * H], 0.0),
                         _NEG_REF)
        return decode(mm, msgs, i == n_steps - 1)

    jax.lax.fori_loop(1, n_steps, step, h0, unroll=False)


def kernel(node_inputs, adj, hints, lengths, W_enc_in, W_enc_hint, W_edge,
           W_m1, W_m2, W_o1, W_o2, W_dec_out, W_dec_hint):
    del W_dec_hint
    B, N, _ = node_inputs.shape
    T = hints.shape[0]
    H = W_o2.shape[1]

    x = node_inputs[..., 0][:, None, :]
    hints_bt = jnp.transpose(hints, (1, 0, 2))
    wdec = W_dec_out[:, 0][None, :]
    wcat = jnp.concatenate([W_m1, W_m2, W_o1], axis=1)

    grid_spec = pltpu.PrefetchScalarGridSpec(
        num_scalar_prefetch=1,
        grid=(B,),
        in_specs=[
            pl.BlockSpec((1, 1, N), lambda b, *_: (b, 0, 0)),
            pl.BlockSpec((1, N, N), lambda b, *_: (b, 0, 0)),
            pl.BlockSpec((1, T, N), lambda b, *_: (b, 0, 0)),
            pl.BlockSpec((1, H), lambda b, *_: (0, 0)),
            pl.BlockSpec((1, H), lambda b, *_: (0, 0)),
            pl.BlockSpec((1, H), lambda b, *_: (0, 0)),
            pl.BlockSpec((2 * H, 3 * H), lambda b, *_: (0, 0)),
            pl.BlockSpec((H, H), lambda b, *_: (0, 0)),
            pl.BlockSpec((1, H), lambda b, *_: (0, 0)),
        ],
        out_specs=pl.BlockSpec((1, 1, N), lambda b, *_: (b, 0, 0)),
        scratch_shapes=[
            pltpu.VMEM((N, N, H), jnp.float32),
        ],
    )

    out = pl.pallas_call(
        _net_step_kernel,
        grid_spec=grid_spec,
        out_shape=jax.ShapeDtypeStruct((B, 1, N), jnp.float32),
    )(lengths, x, adj, hints_bt, W_enc_in, W_enc_hint, W_edge,
      wcat, W_o2, wdec)
    return out[:, 0, :]


# speedup vs baseline: 1.0876x; 1.0876x over previous
import functools

import jax
import jax.numpy as jnp
from jax.experimental import pallas as pl
from jax.experimental.pallas import tpu as pltpu

_NEG_BIG = -3e38
_NEG_REF = -1e9


def _net_step_kernel(lengths_ref, x_ref, adj_ref, hints_ref, we_in_ref,
                     we_hint_ref, w_edge_ref, wcat_ref, wo2_ref, wdec_ref,
                     out_ref, base_ref):
    N = adj_ref.shape[1]
    H = wo2_ref.shape[1]

    b = pl.program_id(0)
    L = lengths_ref[b]
    n_steps = jnp.maximum(L - 1, 1)

    x = x_ref[0, 0, :]
    wedge = w_edge_ref[0, :]
    wdec = wdec_ref[0, :]
    wcat_a, wcat_b = wcat_ref[:H, :], wcat_ref[H:, :]
    wo2 = wo2_ref[...]

    dot = functools.partial(jnp.dot, preferred_element_type=jnp.float32)
    u_wa = dot(we_in_ref[...], wcat_a)
    v_wa = dot(we_hint_ref[...], wcat_a)

    def enc_mm(hint_i):
        hb = jnp.broadcast_to(hint_i[:, None], (N, H))
        return xu + jnp.concatenate(
            [hb * v_wa[:, :H], hb * v_wa[:, H:2 * H], hb * v_wa[:, 2 * H:]],
            axis=1)

    adj = adj_ref[0]
    anycol = jnp.max(adj.T, axis=1, keepdims=True) > 0.5
    xu = x[:, None] * u_wa

    def tail(g, mm):
        msgs = jnp.where(anycol, jnp.maximum(g + mm[:, H:2 * H], 0.0),
                         _NEG_REF)
        return jnp.maximum(mm[:, 2 * H:] + dot(msgs, wo2), 0.0)

    hint0 = hints_ref[0, 0:1, :][0]
    mm0 = enc_mm(hint0)
    m1_0 = mm0[:, :H]
    g = None
    for k in range(0, N, 16):
        a3 = adj[k:k + 16, :][:, :, None]
        blk = jnp.where(a3 > 0.5, a3 * wedge[None, None, :], _NEG_BIG)
        base_ref[k:k + 16] = blk
        for s in range(16):
            cand = m1_0[k + s:k + s + 1, :] + blk[s]
            g = cand if g is None else jnp.maximum(g, cand)

    def step(i, carry):
        g_prev, mm_prev = carry
        hidden = tail(g_prev, mm_prev)
        hint_i = hints_ref[0, pl.ds(i, 1), :][0]
        mm = enc_mm(hint_i) + dot(hidden, wcat_b)
        m1 = mm[:, :H]
        g = m1[0:1, :] + base_ref[0]
        for i_src in range(1, N):
            g = jnp.maximum(g, m1[i_src:i_src + 1, :] + base_ref[i_src])
        return g, mm

    g_f, mm_f = jax.lax.fori_loop(1, n_steps, step, (g, mm0), unroll=False)
    h_final = tail(g_f, mm_f)
    out_ref[0, 0, :] = jnp.sum(h_final * wdec[None, :], axis=1)


def kernel(node_inputs, adj, hints, lengths, W_enc_in, W_enc_hint, W_edge,
           W_m1, W_m2, W_o1, W_o2, W_dec_out, W_dec_hint):
    del W_dec_hint
    B, N, _ = node_inputs.shape
    T = hints.shape[0]
    H = W_o2.shape[1]

    x = node_inputs[..., 0][:, None, :]
    hints_bt = jnp.transpose(hints, (1, 0, 2))
    wdec = W_dec_out[:, 0][None, :]
    wcat = jnp.concatenate([W_m1, W_m2, W_o1], axis=1)

    grid_spec = pltpu.PrefetchScalarGridSpec(
        num_scalar_prefetch=1,
        grid=(B,),
        in_specs=[
            pl.BlockSpec((1, 1, N), lambda b, *_: (b, 0, 0)),
            pl.BlockSpec((1, N, N), lambda b, *_: (b, 0, 0)),
            pl.BlockSpec((1, T, N), lambda b, *_: (b, 0, 0)),
            pl.BlockSpec((1, H), lambda b, *_: (0, 0)),
            pl.BlockSpec((1, H), lambda b, *_: (0, 0)),
            pl.BlockSpec((1, H), lambda b, *_: (0, 0)),
            pl.BlockSpec((2 * H, 3 * H), lambda b, *_: (0, 0)),
            pl.BlockSpec((H, H), lambda b, *_: (0, 0)),
            pl.BlockSpec((1, H), lambda b, *_: (0, 0)),
        ],
        out_specs=pl.BlockSpec((1, 1, N), lambda b, *_: (b, 0, 0)),
        scratch_shapes=[
            pltpu.VMEM((N, N, H), jnp.float32),
        ],
    )

    out = pl.pallas_call(
        _net_step_kernel,
        grid_spec=grid_spec,
        out_shape=jax.ShapeDtypeStruct((B, 1, N), jnp.float32),
    )(lengths, x, adj, hints_bt, W_enc_in, W_enc_hint, W_edge,
      wcat, W_o2, wdec)
    return out[:, 0, :]
